# Initial kernel scaffold; baseline (speedup 1.0000x reference)
import functools

import jax
import jax.numpy as jnp
from jax import lax
from jax.experimental import pallas as pl
from jax.experimental.pallas import tpu as pltpu
from jax.experimental.pallas import tpu_sc as plsc

NC = 2
NS = 16
NW = NC * NS
F = 16
IDX_MINOR = 128
CH = 16


@functools.lru_cache(maxsize=None)
def _build_scatter(num_nodes: int, rows_per_worker: int):
    steps = rows_per_worker // CH
    epc = CH * IDX_MINOR
    rows_per_tile = num_nodes // NS
    zrows = 1250
    n_z = rows_per_tile // zrows

    mesh = plsc.VectorSubcoreMesh(core_axis_name="c", subcore_axis_name="s")

    @functools.partial(
        pl.kernel,
        mesh=mesh,
        out_type=jax.ShapeDtypeStruct((NC * num_nodes, F), jnp.float32),
        scratch_types=[
            pltpu.VMEM((epc, F), jnp.float32),
            pltpu.VMEM((CH, IDX_MINOR), jnp.int32),
            pltpu.VMEM((zrows, F), jnp.float32),
            pltpu.VMEM_SHARED((num_nodes, F), jnp.float32),
        ],
    )
    def scatter_partials(edges_hbm, recv_hbm, out_hbm, ebuf, ibuf, zbuf, acc):
        c = lax.axis_index("c")
        s = lax.axis_index("s")
        wid = c * NS + s

        def zero_body(i, carry):
            zbuf[i, :] = jnp.zeros((F,), jnp.float32)
            return carry

        lax.fori_loop(0, zrows, zero_body, 0)
        for z in range(n_z):
            pltpu.sync_copy(
                zbuf, acc.at[pl.ds(s * rows_per_tile + z * zrows, zrows)]
            )
        plsc.subcore_barrier()

        row_base = wid * rows_per_worker

        def step(t, carry):
            r0 = row_base + t * CH
            pltpu.sync_copy(edges_hbm.at[pl.ds(r0 * IDX_MINOR, epc)], ebuf)
            pltpu.sync_copy(recv_hbm.at[pl.ds(r0, CH)], ibuf)
            for j in range(CH):
                pltpu.sync_copy(
                    ebuf.at[pl.ds(j * IDX_MINOR, IDX_MINOR)],
                    acc.at[ibuf.at[j]],
                    add=True,
                )
            return carry

        lax.fori_loop(0, steps, step, 0)
        plsc.subcore_barrier()

        out_base = c * num_nodes + s * rows_per_tile
        for z in range(n_z):
            pltpu.sync_copy(
                acc.at[pl.ds(s * rows_per_tile + z * zrows, zrows)], zbuf
            )
            pltpu.sync_copy(zbuf, out_hbm.at[pl.ds(out_base + z * zrows, zrows)])

    return scatter_partials


def _merge_body(a_ref, b_ref, o_ref):
    o_ref[...] = a_ref[...] + b_ref[...]


def _merge(p0, p1):
    rows = p0.shape[0]
    br = 1250
    return pl.pallas_call(
        _merge_body,
        out_shape=jax.ShapeDtypeStruct((rows, 128), jnp.float32),
        grid=(rows // br,),
        in_specs=[
            pl.BlockSpec((br, 128), lambda i: (i, 0)),
            pl.BlockSpec((br, 128), lambda i: (i, 0)),
        ],
        out_specs=pl.BlockSpec((br, 128), lambda i: (i, 0)),
    )(p0, p1)


def kernel(nodes, edges, receivers):
    num_nodes = nodes.shape[0]
    num_edges, f = edges.shape

    step_edges = NW * IDX_MINOR * CH
    e_pad = -(-num_edges // step_edges) * step_edges
    pad = e_pad - num_edges
    recv = receivers.astype(jnp.int32)
    edges_p = jnp.concatenate(
        [edges, jnp.zeros((pad, f), edges.dtype)], axis=0
    )
    recv_p = jnp.concatenate(
        [recv, jnp.arange(pad, dtype=jnp.int32) % num_nodes], axis=0
    )
    recv2d = recv_p.reshape(e_pad // IDX_MINOR, IDX_MINOR)

    rows_per_worker = e_pad // (NW * IDX_MINOR)
    partials = _build_scatter(num_nodes, rows_per_worker)(edges_p, recv2d)
    pflat = partials.reshape(NC, num_nodes * f // 128, 128)
    out = _merge(pflat[0], pflat[1])
    return out.reshape(num_nodes, f)


# speedup vs baseline: 3.5223x; 3.5223x over previous
import functools

import jax
import jax.numpy as jnp
from jax import lax
from jax.experimental import pallas as pl
from jax.experimental.pallas import tpu as pltpu
from jax.experimental.pallas import tpu_sc as plsc

NC = 2
NS = 16
NW = NC * NS
F = 16
IDX_MINOR = 128
CH = 8


def _chunk_rows(rows_per_tile: int, epc: int) -> int:
    for d in range(min(epc, rows_per_tile), 7, -1):
        if rows_per_tile % d == 0 and d % 8 == 0:
            return d
    return 8


@functools.lru_cache(maxsize=None)
def _build_scatter(num_nodes: int, rows_per_worker: int):
    steps = rows_per_worker // CH
    epc = CH * IDX_MINOR
    rows_per_tile = num_nodes // NS
    zrows = _chunk_rows(rows_per_tile, epc)
    n_z = rows_per_tile // zrows

    mesh = plsc.VectorSubcoreMesh(core_axis_name="c", subcore_axis_name="s")

    @functools.partial(
        pl.kernel,
        mesh=mesh,
        compiler_params=pltpu.CompilerParams(use_tc_tiling_on_sc=False),
        out_type=jax.ShapeDtypeStruct((NC * num_nodes, F), jnp.float32),
        scratch_types=[
            pltpu.VMEM((epc, F), jnp.float32),
            pltpu.VMEM((CH, IDX_MINOR), jnp.int32),
            pltpu.VMEM_SHARED((num_nodes, F), jnp.float32),
        ],
    )
    def scatter_partials(edges_hbm, recv_hbm, out_hbm, ebuf, ibuf, acc):
        c = lax.axis_index("c")
        s = lax.axis_index("s")
        wid = c * NS + s

        def zero_body(i, carry):
            ebuf[i, :] = jnp.zeros((F,), jnp.float32)
            return carry

        lax.fori_loop(0, zrows, zero_body, 0)
        for z in range(n_z):
            pltpu.sync_copy(
                ebuf.at[pl.ds(0, zrows)],
                acc.at[pl.ds(s * rows_per_tile + z * zrows, zrows)],
            )
        plsc.subcore_barrier()

        row_base = wid * rows_per_worker

        def step(t, carry):
            r0 = row_base + t * CH
            pltpu.sync_copy(edges_hbm.at[pl.ds(r0 * IDX_MINOR, epc)], ebuf)
            pltpu.sync_copy(recv_hbm.at[pl.ds(r0, CH)], ibuf)
            for j in range(CH):
                pltpu.sync_copy(
                    ebuf.at[pl.ds(j * IDX_MINOR, IDX_MINOR)],
                    acc.at[ibuf.at[j]],
                    add=True,
                )
            return carry

        lax.fori_loop(0, steps, step, 0)
        plsc.subcore_barrier()

        out_base = c * num_nodes + s * rows_per_tile
        for z in range(n_z):
            pltpu.sync_copy(
                acc.at[pl.ds(s * rows_per_tile + z * zrows, zrows)],
                ebuf.at[pl.ds(0, zrows)],
            )
            pltpu.sync_copy(
                ebuf.at[pl.ds(0, zrows)],
                out_hbm.at[pl.ds(out_base + z * zrows, zrows)],
            )

    return scatter_partials


def _merge_body(a_ref, b_ref, o_ref):
    o_ref[...] = a_ref[...] + b_ref[...]


def _merge(p0, p1):
    rows = p0.shape[0]
    return pl.pallas_call(
        _merge_body,
        out_shape=jax.ShapeDtypeStruct((rows, 128), jnp.float32),
    )(p0, p1)


def kernel(nodes, edges, receivers):
    num_nodes = nodes.shape[0]
    num_edges, f = edges.shape

    step_edges = NW * IDX_MINOR * CH
    e_pad = -(-num_edges // step_edges) * step_edges
    pad = e_pad - num_edges
    recv = receivers.astype(jnp.int32)
    edges_p = jnp.concatenate(
        [edges, jnp.zeros((pad, f), edges.dtype)], axis=0
    )
    recv_p = jnp.concatenate(
        [recv, jnp.arange(pad, dtype=jnp.int32) % num_nodes], axis=0
    )
    recv2d = recv_p.reshape(e_pad // IDX_MINOR, IDX_MINOR)

    n_align = NS * 32
    n_pad = -(-num_nodes // n_align) * n_align
    rows_per_worker = e_pad // (NW * IDX_MINOR)
    partials = _build_scatter(n_pad, rows_per_worker)(edges_p, recv2d)
    pflat = partials.reshape(NC, n_pad * f // 128, 128)
    out = _merge(pflat[0], pflat[1])
    return out.reshape(n_pad, f)[:num_nodes]


# speedup vs baseline: 5.6435x; 1.6022x over previous
import functools

import jax
import jax.numpy as jnp
from jax import lax
from jax.experimental import pallas as pl
from jax.experimental.pallas import tpu as pltpu
from jax.experimental.pallas import tpu_sc as plsc

NC = 2
NS = 16
NW = NC * NS
F = 16
IDX_MINOR = 128
CH = 8


def _chunk_rows(rows_per_tile: int, epc: int) -> int:
    for d in range(min(epc, rows_per_tile), 7, -1):
        if rows_per_tile % d == 0 and d % 8 == 0:
            return d
    return 8


@functools.lru_cache(maxsize=None)
def _build_scatter(num_nodes: int, total_windows: int):
    wq, wr = divmod(total_windows, NW)
    epc = CH * IDX_MINOR
    rows_per_tile = num_nodes // NS
    zrows = _chunk_rows(rows_per_tile, epc)
    n_z = rows_per_tile // zrows

    mesh = plsc.VectorSubcoreMesh(core_axis_name="c", subcore_axis_name="s")

    @functools.partial(
        pl.kernel,
        mesh=mesh,
        compiler_params=pltpu.CompilerParams(use_tc_tiling_on_sc=False),
        out_type=jax.ShapeDtypeStruct((NC * num_nodes, F), jnp.float32),
        scratch_types=[
            pltpu.VMEM((epc, F), jnp.float32),
            pltpu.VMEM((CH, IDX_MINOR), jnp.int32),
            pltpu.VMEM_SHARED((num_nodes, F), jnp.float32),
        ],
    )
    def scatter_partials(edges_hbm, recv_hbm, out_hbm, ebuf, ibuf, acc):
        c = lax.axis_index("c")
        s = lax.axis_index("s")
        wid = c * NS + s

        def zero_body(i, carry):
            ebuf[i, :] = jnp.zeros((F,), jnp.float32)
            return carry

        lax.fori_loop(0, zrows, zero_body, 0)
        for z in range(n_z):
            pltpu.sync_copy(
                ebuf.at[pl.ds(0, zrows)],
                acc.at[pl.ds(s * rows_per_tile + z * zrows, zrows)],
            )
        plsc.subcore_barrier()

        steps = wq + jnp.where(wid < wr, 1, 0)
        win_base = wq * wid + jnp.minimum(wid, wr)

        def step(t, carry):
            r0 = (win_base + t) * CH
            pltpu.sync_copy(edges_hbm.at[pl.ds(r0 * IDX_MINOR, epc)], ebuf)
            pltpu.sync_copy(recv_hbm.at[pl.ds(r0, CH)], ibuf)
            for j in range(CH):
                pltpu.sync_copy(
                    ebuf.at[pl.ds(j * IDX_MINOR, IDX_MINOR)],
                    acc.at[ibuf.at[j]],
                    add=True,
                )
            return carry

        lax.fori_loop(0, steps, step, 0)
        plsc.subcore_barrier()

        out_base = c * num_nodes + s * rows_per_tile
        for z in range(n_z):
            pltpu.sync_copy(
                acc.at[pl.ds(s * rows_per_tile + z * zrows, zrows)],
                ebuf.at[pl.ds(0, zrows)],
            )
            pltpu.sync_copy(
                ebuf.at[pl.ds(0, zrows)],
                out_hbm.at[pl.ds(out_base + z * zrows, zrows)],
            )

    return scatter_partials


def _merge_body(a_ref, b_ref, o_ref):
    o_ref[...] = a_ref[...] + b_ref[...]


def _merge(p0, p1):
    rows = p0.shape[0]
    return pl.pallas_call(
        _merge_body,
        out_shape=jax.ShapeDtypeStruct((rows, 128), jnp.float32),
    )(p0, p1)


def kernel(nodes, edges, receivers):
    num_nodes = nodes.shape[0]
    num_edges, f = edges.shape

    step_edges = IDX_MINOR * CH
    e_pad = -(-num_edges // step_edges) * step_edges
    pad = e_pad - num_edges
    recv = receivers.astype(jnp.int32)
    if pad:
        edges = jnp.concatenate(
            [edges, jnp.zeros((pad, f), edges.dtype)], axis=0
        )
        recv = jnp.concatenate(
            [recv, jnp.arange(pad, dtype=jnp.int32) % num_nodes], axis=0
        )
    recv2d = recv.reshape(e_pad // IDX_MINOR, IDX_MINOR)

    n_align = NS * 32
    n_pad = -(-num_nodes // n_align) * n_align
    total_windows = e_pad // step_edges
    partials = _build_scatter(n_pad, total_windows)(edges, recv2d)
    pflat = partials.reshape(NC, n_pad * f // 128, 128)
    out = _merge(pflat[0], pflat[1])
    return out.reshape(n_pad, f)[:num_nodes]
